# baseline (speedup 1.0000x reference)
import functools

import jax
import jax.numpy as jnp
from jax import lax
from jax.experimental import pallas as pl
from jax.experimental.pallas import tpu as pltpu
from jax.experimental.pallas import tpu_sc as plsc

_R = 4


def _make_embed(B: int, L: int, vocab: int, dim: int):
    info = plsc.get_sparse_core_info()
    nw = info.num_cores * info.num_subcores
    assert B % (nw * _R) == 0 and L % 8 == 0
    b_per_w = B // nw
    i_per_w = b_per_w * L
    n_outer = b_per_w // _R
    chunks = []
    off = 0
    while off < L:
        n = min(128, L - off)
        while n % 8:
            n -= 1
        chunks.append((off, n))
        off += n

    mesh = plsc.VectorSubcoreMesh(core_axis_name="c", subcore_axis_name="s")

    @functools.partial(
        pl.kernel,
        mesh=mesh,
        compiler_params=pltpu.CompilerParams(use_tc_tiling_on_sc=False),
        out_type=jax.ShapeDtypeStruct((B, L, dim), jnp.float32),
        scratch_types=[
            pltpu.VMEM((i_per_w,), jnp.int32),
            pltpu.VMEM((_R, L, dim), jnp.float32),
            pltpu.SemaphoreType.DMA,
        ] + [pltpu.SemaphoreType.DMA] * _R,
    )
    def embed(table_hbm, idx_hbm, out_hbm, idx_v, rows_v, isem, *sems):
        wid = lax.axis_index("s") * info.num_cores + lax.axis_index("c")
        b0 = wid * b_per_w
        ibase = pl.multiple_of(wid * i_per_w, i_per_w)

        pltpu.async_copy(idx_hbm.at[pl.ds(ibase, i_per_w)], idx_v, isem).wait()

        def gather(b, s):
            ib = pl.multiple_of(b * L, 8)
            for (o, n) in chunks:
                pltpu.async_copy(
                    table_hbm.at[idx_v.at[pl.ds(ib + o, n)]],
                    rows_v.at[s, pl.ds(o, n)], sems[s])

        def wait_slot(s):
            pltpu.make_async_copy(
                table_hbm.at[pl.ds(0, L)], rows_v.at[s], sems[s]).wait()

        def store(b, s):
            pltpu.async_copy(rows_v.at[s], out_hbm.at[b0 + b], sems[s])

        for s in range(_R):
            gather(s, s)

        def body(g, _):
            for s in range(_R):
                b = g * _R + s
                wait_slot(s)
                store(b, s)
                @pl.when(g < n_outer - 1)
                def _():
                    wait_slot(s)
                    gather(b + _R, s)
            return 0

        lax.fori_loop(0, n_outer, body, 0, unroll=False)

        for s in range(_R):
            wait_slot(s)

    return embed


def kernel(x, W):
    B, L = x.shape
    V, D = W.shape
    embed = _make_embed(B, L, V, D)
    return embed(W, x.reshape(B * L))


# speedup vs baseline: 1.0008x; 1.0008x over previous
import functools

import jax
import jax.numpy as jnp
from jax import lax
from jax.experimental import pallas as pl
from jax.experimental.pallas import tpu as pltpu
from jax.experimental.pallas import tpu_sc as plsc

_R = 8


def _make_embed(B: int, L: int, vocab: int, dim: int):
    info = plsc.get_sparse_core_info()
    nw = info.num_cores * info.num_subcores
    assert B % (nw * _R) == 0 and L % 8 == 0
    b_per_w = B // nw
    i_per_w = b_per_w * L
    n_outer = b_per_w // _R
    chunks = []
    off = 0
    while off < L:
        n = min(128, L - off)
        while n % 8:
            n -= 1
        chunks.append((off, n))
        off += n

    mesh = plsc.VectorSubcoreMesh(core_axis_name="c", subcore_axis_name="s")

    @functools.partial(
        pl.kernel,
        mesh=mesh,
        compiler_params=pltpu.CompilerParams(use_tc_tiling_on_sc=False),
        out_type=jax.ShapeDtypeStruct((B, L, dim), jnp.float32),
        scratch_types=[
            pltpu.VMEM((i_per_w,), jnp.int32),
            pltpu.VMEM((_R, L, dim), jnp.float32),
            pltpu.SemaphoreType.DMA,
        ] + [pltpu.SemaphoreType.DMA] * _R,
    )
    def embed(table_hbm, idx_hbm, out_hbm, idx_v, rows_v, isem, *sems):
        wid = lax.axis_index("s") * info.num_cores + lax.axis_index("c")
        b0 = wid * b_per_w
        ibase = pl.multiple_of(wid * i_per_w, i_per_w)

        pltpu.async_copy(idx_hbm.at[pl.ds(ibase, i_per_w)], idx_v, isem).wait()

        def gather(b, s):
            ib = pl.multiple_of(b * L, 8)
            for (o, n) in chunks:
                pltpu.async_copy(
                    table_hbm.at[idx_v.at[pl.ds(ib + o, n)]],
                    rows_v.at[s, pl.ds(o, n)], sems[s])

        def wait_slot(s):
            pltpu.make_async_copy(
                table_hbm.at[pl.ds(0, L)], rows_v.at[s], sems[s]).wait()

        def store(b, s):
            pltpu.async_copy(rows_v.at[s], out_hbm.at[b0 + b], sems[s])

        for s in range(_R):
            gather(s, s)

        def body(g, _):
            for s in range(_R):
                b = g * _R + s
                wait_slot(s)
                store(b, s)
                @pl.when(g < n_outer - 1)
                def _():
                    wait_slot(s)
                    gather(b + _R, s)
            return 0

        lax.fori_loop(0, n_outer, body, 0, unroll=False)

        for s in range(_R):
            wait_slot(s)

    return embed


def kernel(x, W):
    B, L = x.shape
    V, D = W.shape
    embed = _make_embed(B, L, V, D)
    return embed(W, x.reshape(B * L))
